# baseline (speedup 1.0000x reference)
import jax
import jax.numpy as jnp
from jax.experimental import pallas as pl
from jax.experimental.pallas import tpu as pltpu

_N_NODES = 39
_HIDDEN = 128
_CONT = 13
_CATF = 26
_NUM_CLASSES = 10


def _build_adjacency(ei_ref, ew_ref, a_ref):
    src = ei_ref[0, :]
    dst = ei_ref[1, :]
    w = ew_ref[0, :]
    e = src.shape[0]
    n = _N_NODES
    node_ids = jax.lax.broadcasted_iota(jnp.int32, (e, n), 1)
    osrc = (src[:, None] == node_ids).astype(jnp.float32)
    odst = (dst[:, None] == node_ids).astype(jnp.float32)
    deg = 1.0 + jnp.sum(odst * w[:, None], axis=0)
    dinv = jnp.where(deg > 0, jax.lax.rsqrt(deg), 0.0)
    dinv_src = jnp.sum(osrc * dinv[None, :], axis=1)
    dinv_dst = jnp.sum(odst * dinv[None, :], axis=1)
    norm = dinv_src * w * dinv_dst
    a = jax.lax.dot_general(
        odst * norm[:, None], osrc, (((0,), (0,)), ((), ())),
        preferred_element_type=jnp.float32)
    rows = jax.lax.broadcasted_iota(jnp.int32, (n, n), 0)
    cols = jax.lax.broadcasted_iota(jnp.int32, (n, n), 1)
    a_ref[...] = a + jnp.where(rows == cols, dinv[:, None] * dinv[None, :], 0.0)


def _body(ei_ref, ew_ref, vanT_ref, fcw_ref, fcb_ref,
          num_ref, cat2_ref, embw_ref, convw_ref,
          outT_ref, h_ref, a_ref):
    i = pl.program_id(0)
    ni = pl.num_programs(0)

    @pl.when(i == 0)
    def _():
        h_ref[0:_CONT, :] = jax.lax.dot_general(
            num_ref[...], convw_ref[...], (((1,), (0,)), ((), ())),
            preferred_element_type=jnp.float32)
        _build_adjacency(ei_ref, ew_ref, a_ref)

    emb = jnp.sum(cat2_ref[...] * embw_ref[0], axis=1, keepdims=True)
    h_ref[pl.ds(_CONT + i, 1), :] = jax.lax.dot_general(
        emb, convw_ref[...], (((0,), (0,)), ((), ())),
        preferred_element_type=jnp.float32)

    @pl.when(i == ni - 1)
    def _():
        hn = jax.nn.relu(
            jax.lax.dot_general(a_ref[...], h_ref[...], (((1,), (0,)), ((), ())),
                                preferred_element_type=jnp.float32))
        pooled = jnp.sum(hn, axis=0, keepdims=True) / jnp.float32(_N_NODES)
        c = jax.lax.dot_general(
            pooled, fcw_ref[_NUM_CLASSES:, :], (((1,), (0,)), ((), ())),
            preferred_element_type=jnp.float32)[0, 0] + fcb_ref[0, 0]
        z = jax.lax.dot_general(
            fcw_ref[:_NUM_CLASSES, :], vanT_ref[...], (((0,), (0,)), ((), ())),
            preferred_element_type=jnp.float32) + c
        beta = jnp.float32(1.1)
        bz = beta * z
        t = (jnp.maximum(bz, 0.0) + jnp.log1p(jnp.exp(-jnp.abs(bz)))) / beta
        outT_ref[...] = vanT_ref[...] / t


@jax.jit
def kernel(num_x, cat_x, edge_index, edge_weights, batch, vanilla_out,
           emb_W, conv_W, fc_W, fc_b):
    del batch
    f, nf, cl = cat_x.shape
    b = vanilla_out.shape[0]
    cat2 = cat_x.reshape(f * nf, cl)
    outT = pl.pallas_call(
        _body,
        grid=(f,),
        in_specs=[
            pl.BlockSpec((2, edge_index.shape[1]), lambda i: (0, 0)),
            pl.BlockSpec((1, edge_weights.shape[0]), lambda i: (0, 0)),
            pl.BlockSpec((_NUM_CLASSES, b), lambda i: (0, 0)),
            pl.BlockSpec(fc_W.shape, lambda i: (0, 0)),
            pl.BlockSpec((1, 1), lambda i: (0, 0)),
            pl.BlockSpec((_CONT, nf), lambda i: (0, 0)),
            pl.BlockSpec((nf, cl), lambda i: (i, 0)),
            pl.BlockSpec((1, 1, cl), lambda i: (i, 0, 0)),
            pl.BlockSpec((nf, _HIDDEN), lambda i: (0, 0)),
        ],
        out_specs=pl.BlockSpec((_NUM_CLASSES, b), lambda i: (0, 0)),
        out_shape=jax.ShapeDtypeStruct((_NUM_CLASSES, b), jnp.float32),
        scratch_shapes=[
            pltpu.VMEM((_N_NODES, _HIDDEN), jnp.float32),
            pltpu.VMEM((_N_NODES, _N_NODES), jnp.float32),
        ],
    )(edge_index, edge_weights.reshape(1, -1), vanilla_out.T, fc_W,
      fc_b.reshape(1, 1), num_x, cat2, emb_W.reshape(f, 1, cl), conv_W)
    return outT.T


# speedup vs baseline: 1.1506x; 1.1506x over previous
import jax
import jax.numpy as jnp
from jax.experimental import pallas as pl
from jax.experimental.pallas import tpu as pltpu

_N_NODES = 39
_HIDDEN = 128
_CONT = 13
_CATF = 26
_NUM_CLASSES = 10


def _build_adjacency(ei_ref, ew_ref, a_ref):
    src = ei_ref[0, :]
    dst = ei_ref[1, :]
    w = ew_ref[0, :]
    e = src.shape[0]
    n = _N_NODES
    node_ids = jax.lax.broadcasted_iota(jnp.int32, (e, n), 1)
    osrc = (src[:, None] == node_ids).astype(jnp.float32)
    odst = (dst[:, None] == node_ids).astype(jnp.float32)
    deg = 1.0 + jnp.sum(odst * w[:, None], axis=0)
    dinv = jnp.where(deg > 0, jax.lax.rsqrt(deg), 0.0)
    dinv_src = jnp.sum(osrc * dinv[None, :], axis=1)
    dinv_dst = jnp.sum(odst * dinv[None, :], axis=1)
    norm = dinv_src * w * dinv_dst
    a = jax.lax.dot_general(
        odst * norm[:, None], osrc, (((0,), (0,)), ((), ())),
        preferred_element_type=jnp.float32)
    rows = jax.lax.broadcasted_iota(jnp.int32, (n, n), 0)
    cols = jax.lax.broadcasted_iota(jnp.int32, (n, n), 1)
    a_ref[...] = a + jnp.where(rows == cols, dinv[:, None] * dinv[None, :], 0.0)


def _body(ei_ref, ew_ref, vanT_ref, fcw_ref, fcb_ref,
          num_ref, cat2_ref, embw_ref, convw_ref,
          outT_ref, h_ref, a_ref):
    i = pl.program_id(0)
    ni = pl.num_programs(0)

    @pl.when(i == 0)
    def _():
        h_ref[0:_CONT, :] = jax.lax.dot_general(
            num_ref[...], convw_ref[...], (((1,), (0,)), ((), ())),
            preferred_element_type=jnp.float32)
        _build_adjacency(ei_ref, ew_ref, a_ref)

    emb = jax.lax.dot_general(
        cat2_ref[...], embw_ref[0, 0][:, None], (((1,), (0,)), ((), ())),
        preferred_element_type=jnp.float32)
    h_ref[pl.ds(_CONT + i, 1), :] = jax.lax.dot_general(
        emb, convw_ref[...], (((0,), (0,)), ((), ())),
        preferred_element_type=jnp.float32)

    @pl.when(i == ni - 1)
    def _():
        hn = jax.nn.relu(
            jax.lax.dot_general(a_ref[...], h_ref[...], (((1,), (0,)), ((), ())),
                                preferred_element_type=jnp.float32))
        pooled = jnp.sum(hn, axis=0, keepdims=True) / jnp.float32(_N_NODES)
        c = jax.lax.dot_general(
            pooled, fcw_ref[_NUM_CLASSES:, :], (((1,), (0,)), ((), ())),
            preferred_element_type=jnp.float32)[0, 0] + fcb_ref[0, 0]
        z = jax.lax.dot_general(
            fcw_ref[:_NUM_CLASSES, :], vanT_ref[...], (((0,), (0,)), ((), ())),
            preferred_element_type=jnp.float32) + c
        beta = jnp.float32(1.1)
        bz = beta * z
        t = (jnp.maximum(bz, 0.0) + jnp.log1p(jnp.exp(-jnp.abs(bz)))) / beta
        outT_ref[...] = vanT_ref[...] / t


@jax.jit
def kernel(num_x, cat_x, edge_index, edge_weights, batch, vanilla_out,
           emb_W, conv_W, fc_W, fc_b):
    del batch
    f, nf, cl = cat_x.shape
    b = vanilla_out.shape[0]
    cat2 = cat_x.reshape(f * nf, cl)
    outT = pl.pallas_call(
        _body,
        grid=(f,),
        in_specs=[
            pl.BlockSpec((2, edge_index.shape[1]), lambda i: (0, 0)),
            pl.BlockSpec((1, edge_weights.shape[0]), lambda i: (0, 0)),
            pl.BlockSpec((_NUM_CLASSES, b), lambda i: (0, 0)),
            pl.BlockSpec(fc_W.shape, lambda i: (0, 0)),
            pl.BlockSpec((1, 1), lambda i: (0, 0)),
            pl.BlockSpec((_CONT, nf), lambda i: (0, 0)),
            pl.BlockSpec((nf, cl), lambda i: (i, 0)),
            pl.BlockSpec((1, 1, cl), lambda i: (i, 0, 0)),
            pl.BlockSpec((nf, _HIDDEN), lambda i: (0, 0)),
        ],
        out_specs=pl.BlockSpec((_NUM_CLASSES, b), lambda i: (0, 0)),
        out_shape=jax.ShapeDtypeStruct((_NUM_CLASSES, b), jnp.float32),
        scratch_shapes=[
            pltpu.VMEM((_N_NODES, _HIDDEN), jnp.float32),
            pltpu.VMEM((_N_NODES, _N_NODES), jnp.float32),
        ],
    )(edge_index, edge_weights.reshape(1, -1), vanilla_out.T, fc_W,
      fc_b.reshape(1, 1), num_x, cat2, emb_W.reshape(f, 1, cl), conv_W)
    return outT.T
